# Initial kernel scaffold; baseline (speedup 1.0000x reference)
import jax
import jax.numpy as jnp
from jax.experimental import pallas as pl


def kernel(feat, edge_index, efeat, W_edge, b_edge, W_ih, W_hh, b_ih, b_hh):
    raise NotImplementedError("write your pallas kernel here")


# speedup vs baseline: 4.3614x; 4.3614x over previous
import functools

import jax
import jax.numpy as jnp
from jax import lax
from jax.experimental import pallas as pl
from jax.experimental.pallas import tpu as pltpu
from jax.experimental.pallas import tpu_sc as plsc

N_NODES = 10000
N_EDGES = 320000
F = 16
FF = F * F
STEPS = 2

GROUP = 80
NW = 32
EPW = N_EDGES // NW
CHUNK = 2000
CPW = EPW // CHUNK
IR = CHUNK // GROUP
BURSTS = IR // 5
NSUB = 16
N_PAD = 10240
NPW = N_PAD // NSUB

BE = 8000


def _sc_mesh():
    return plsc.VectorSubcoreMesh(core_axis_name="c", subcore_axis_name="s")


def _sc_gather(h, src2d):
    @functools.partial(
        pl.kernel,
        mesh=_sc_mesh(),
        out_type=jax.ShapeDtypeStruct((N_EDGES, F), jnp.float32),
        scratch_types=[
            pltpu.VMEM((IR, GROUP), jnp.int32),
            pltpu.VMEM((CHUNK, F), jnp.float32),
            pltpu.SemaphoreType.DMA,
        ],
        compiler_params=pltpu.CompilerParams(use_tc_tiling_on_sc=False),
    )
    def k(h_hbm, src_hbm, out_hbm, idx_v, rows_v, sem):
        wid = lax.axis_index("s") * 2 + lax.axis_index("c")

        def chunk_body(ci, carry):
            pltpu.sync_copy(src_hbm.at[wid, ci], idx_v)

            def burst(bi, c2):
                cps = [
                    pltpu.async_copy(
                        h_hbm.at[idx_v.at[bi * 5 + g]],
                        rows_v.at[pl.ds((bi * 5 + g) * GROUP, GROUP)],
                        sem,
                    )
                    for g in range(5)
                ]
                for cp in cps:
                    cp.wait()
                return c2

            lax.fori_loop(0, BURSTS, burst, 0)
            e0 = wid * EPW + ci * CHUNK
            pltpu.sync_copy(rows_v, out_hbm.at[pl.ds(e0, CHUNK)])
            return carry

        lax.fori_loop(0, CPW, chunk_body, 0)

    return k(h, src2d)


def _sc_scatter(m, dst2d, zeros):
    @functools.partial(
        pl.kernel,
        mesh=_sc_mesh(),
        out_type=jax.ShapeDtypeStruct((2, N_PAD, F), jnp.float32),
        scratch_types=[
            pltpu.VMEM((IR, GROUP), jnp.int32),
            pltpu.VMEM((CHUNK, F), jnp.float32),
            pltpu.VMEM_SHARED((N_PAD, F), jnp.float32),
        ],
        compiler_params=pltpu.CompilerParams(use_tc_tiling_on_sc=False),
    )
    def k(m_hbm, dst_hbm, z_hbm, out_hbm, idx_v, rows_v, acc):
        cid = lax.axis_index("c")
        sid = lax.axis_index("s")
        wid = sid * 2 + cid

        pltpu.sync_copy(z_hbm.at[pl.ds(sid * NPW, NPW)],
                        acc.at[pl.ds(sid * NPW, NPW)])
        plsc.subcore_barrier()

        def chunk_body(ci, carry):
            pltpu.sync_copy(dst_hbm.at[wid, ci], idx_v)
            e0 = wid * EPW + ci * CHUNK
            pltpu.sync_copy(m_hbm.at[pl.ds(e0, CHUNK)], rows_v)

            def burst(bi, c2):
                for g in range(5):
                    jj = bi * 5 + g
                    pltpu.sync_copy(
                        rows_v.at[pl.ds(jj * GROUP, GROUP)],
                        acc.at[idx_v.at[jj]],
                        add=True,
                    )
                return c2

            lax.fori_loop(0, BURSTS, burst, 0)
            return carry

        lax.fori_loop(0, CPW, chunk_body, 0)
        plsc.subcore_barrier()

        pltpu.sync_copy(acc.at[pl.ds(sid * NPW, NPW)],
                        out_hbm.at[cid, pl.ds(sid * NPW, NPW)])

    return k(m, dst2d, zeros)


def _edge_body(hs_ref, ef_ref, we_ref, be_ref, r_ref, t_ref, out_ref):
    w = jnp.dot(ef_ref[...], we_ref[...],
                preferred_element_type=jnp.float32) + be_ref[...]
    hx = jnp.dot(hs_ref[...], r_ref[...], preferred_element_type=jnp.float32)
    out_ref[...] = jnp.dot(hx * w, t_ref[...],
                           preferred_element_type=jnp.float32)


def _tc_edge(h_src, efeat, W_edge, b_edge2, R, T):
    return pl.pallas_call(
        _edge_body,
        grid=(N_EDGES // BE,),
        in_specs=[
            pl.BlockSpec((BE, F), lambda i: (i, 0)),
            pl.BlockSpec((BE, F), lambda i: (i, 0)),
            pl.BlockSpec((F, FF), lambda i: (0, 0)),
            pl.BlockSpec((1, FF), lambda i: (0, 0)),
            pl.BlockSpec((F, FF), lambda i: (0, 0)),
            pl.BlockSpec((FF, F), lambda i: (0, 0)),
        ],
        out_specs=pl.BlockSpec((BE, F), lambda i: (i, 0)),
        out_shape=jax.ShapeDtypeStruct((N_EDGES, F), jnp.float32),
    )(h_src, efeat, W_edge, b_edge2, R, T)


def _gru_body(p_ref, h_ref, wi_ref, wh_ref, bi_ref, bh_ref, out_ref):
    x = p_ref[0, :N_NODES, :] + p_ref[1, :N_NODES, :]
    h = h_ref[...]
    gi = jnp.dot(x, wi_ref[...], preferred_element_type=jnp.float32) + bi_ref[...]
    gh = jnp.dot(h, wh_ref[...], preferred_element_type=jnp.float32) + bh_ref[...]
    r = jax.nn.sigmoid(gi[:, 0:F] + gh[:, 0:F])
    z = jax.nn.sigmoid(gi[:, F:2 * F] + gh[:, F:2 * F])
    n = jnp.tanh(gi[:, 2 * F:3 * F] + r * gh[:, 2 * F:3 * F])
    out_ref[...] = (1.0 - z) * n + z * h


def _tc_gru(parts, h, WihT, WhhT, bih2, bhh2):
    return pl.pallas_call(
        _gru_body,
        out_shape=jax.ShapeDtypeStruct((N_NODES, F), jnp.float32),
    )(parts, h, WihT, WhhT, bih2, bhh2)


def kernel(feat, edge_index, efeat, W_edge, b_edge, W_ih, W_hh, b_ih, b_hh):
    src2d = edge_index[0].reshape(NW, CPW, IR, GROUP)
    dst2d = edge_index[1].reshape(NW, CPW, IR, GROUP)
    b_edge2 = b_edge.reshape(1, FF)
    eye = jnp.eye(F, dtype=jnp.float32)
    R = jnp.repeat(eye, F, axis=1)
    T = jnp.tile(eye, (F, 1))
    WihT = W_ih.T
    WhhT = W_hh.T
    bih2 = b_ih.reshape(1, 3 * F)
    bhh2 = b_hh.reshape(1, 3 * F)
    zeros = jnp.zeros((N_PAD, F), jnp.float32)

    h = feat
    for _ in range(STEPS):
        h_src = _sc_gather(h, src2d)
        m = _tc_edge(h_src, efeat, W_edge, b_edge2, R, T)
        parts = _sc_scatter(m, dst2d, zeros)
        h = _tc_gru(parts, h, WihT, WhhT, bih2, bhh2)
    return h


# speedup vs baseline: 7.4133x; 1.6997x over previous
import functools

import jax
import jax.numpy as jnp
from jax import lax
from jax.experimental import pallas as pl
from jax.experimental.pallas import tpu as pltpu
from jax.experimental.pallas import tpu_sc as plsc

N_NODES = 10000
N_EDGES = 320000
F = 16
FF = F * F
STEPS = 2

GROUP = 80
NW = 32
EPW = N_EDGES // NW
CHUNK = 2000
CPW = EPW // CHUNK
IR = CHUNK // GROUP
BURSTS = IR // 5
NSUB = 16
N_PAD = 10240
NPW = N_PAD // NSUB

BE = 8000


def _sc_mesh():
    return plsc.VectorSubcoreMesh(core_axis_name="c", subcore_axis_name="s")


def _sc_gather(h, src2d):
    @functools.partial(
        pl.kernel,
        mesh=_sc_mesh(),
        out_type=jax.ShapeDtypeStruct((N_EDGES, F), jnp.float32),
        scratch_types=[
            pltpu.VMEM((IR, GROUP), jnp.int32),
            pltpu.VMEM((CHUNK, F), jnp.float32),
            pltpu.SemaphoreType.DMA,
        ],
        compiler_params=pltpu.CompilerParams(use_tc_tiling_on_sc=False),
    )
    def k(h_hbm, src_hbm, out_hbm, idx_v, rows_v, sem):
        wid = lax.axis_index("s") * 2 + lax.axis_index("c")

        def chunk_body(ci, carry):
            pltpu.sync_copy(src_hbm.at[wid, ci], idx_v)

            def burst(bi, c2):
                cps = [
                    pltpu.async_copy(
                        h_hbm.at[idx_v.at[bi * 5 + g]],
                        rows_v.at[pl.ds((bi * 5 + g) * GROUP, GROUP)],
                        sem,
                    )
                    for g in range(5)
                ]
                for cp in cps:
                    cp.wait()
                return c2

            lax.fori_loop(0, BURSTS, burst, 0)
            e0 = wid * EPW + ci * CHUNK
            pltpu.sync_copy(rows_v, out_hbm.at[pl.ds(e0, CHUNK)])
            return carry

        lax.fori_loop(0, CPW, chunk_body, 0)

    return k(h, src2d)


def _sc_scatter(m, dst2d, zeros):
    @functools.partial(
        pl.kernel,
        mesh=_sc_mesh(),
        out_type=jax.ShapeDtypeStruct((2, N_PAD, F), jnp.float32),
        scratch_types=[
            pltpu.VMEM((IR, GROUP), jnp.int32),
            pltpu.VMEM((CHUNK, F), jnp.float32),
            pltpu.VMEM_SHARED((N_PAD, F), jnp.float32),
        ],
        compiler_params=pltpu.CompilerParams(use_tc_tiling_on_sc=False),
    )
    def k(m_hbm, dst_hbm, z_hbm, out_hbm, idx_v, rows_v, acc):
        cid = lax.axis_index("c")
        sid = lax.axis_index("s")
        wid = sid * 2 + cid

        pltpu.sync_copy(z_hbm.at[pl.ds(sid * NPW, NPW)],
                        acc.at[pl.ds(sid * NPW, NPW)])
        plsc.subcore_barrier()

        def chunk_body(ci, carry):
            pltpu.sync_copy(dst_hbm.at[wid, ci], idx_v)
            e0 = wid * EPW + ci * CHUNK
            pltpu.sync_copy(m_hbm.at[pl.ds(e0, CHUNK)], rows_v)

            def burst(bi, c2):
                for g in range(5):
                    jj = bi * 5 + g
                    pltpu.sync_copy(
                        rows_v.at[pl.ds(jj * GROUP, GROUP)],
                        acc.at[idx_v.at[jj]],
                        add=True,
                    )
                return c2

            lax.fori_loop(0, BURSTS, burst, 0)
            return carry

        lax.fori_loop(0, CPW, chunk_body, 0)
        plsc.subcore_barrier()

        pltpu.sync_copy(acc.at[pl.ds(sid * NPW, NPW)],
                        out_hbm.at[cid, pl.ds(sid * NPW, NPW)])

    return k(m, dst2d, zeros)


BR = BE // 8


def _edge_body(hs_ref, ef_ref, we_ref, be_ref, r_ref, t_ref, out_ref):
    we = we_ref[...]
    be = be_ref[...]
    r = r_ref[...]
    t = t_ref[...]
    for j in range(8):
        hsj = hs_ref[:, j * F:(j + 1) * F]
        efj = ef_ref[:, j * F:(j + 1) * F]
        w = jnp.dot(efj, we, preferred_element_type=jnp.float32) + be
        hx = jnp.dot(hsj, r, preferred_element_type=jnp.float32)
        out_ref[:, j * F:(j + 1) * F] = jnp.dot(
            hx * w, t, preferred_element_type=jnp.float32)


def _tc_edge(h_srcp, efeatp, W_edge, b_edge2, R, T):
    return pl.pallas_call(
        _edge_body,
        grid=(N_EDGES // BE,),
        in_specs=[
            pl.BlockSpec((BR, 128), lambda i: (i, 0)),
            pl.BlockSpec((BR, 128), lambda i: (i, 0)),
            pl.BlockSpec((F, FF), lambda i: (0, 0)),
            pl.BlockSpec((1, FF), lambda i: (0, 0)),
            pl.BlockSpec((F, FF), lambda i: (0, 0)),
            pl.BlockSpec((FF, F), lambda i: (0, 0)),
        ],
        out_specs=pl.BlockSpec((BR, 128), lambda i: (i, 0)),
        out_shape=jax.ShapeDtypeStruct((N_EDGES // 8, 128), jnp.float32),
    )(h_srcp, efeatp, W_edge, b_edge2, R, T)


def _gru_body(p_ref, h_ref, wi_ref, wh_ref, bi_ref, bh_ref, out_ref):
    x = p_ref[0, :N_NODES, :] + p_ref[1, :N_NODES, :]
    h = h_ref[...]
    gi = jnp.dot(x, wi_ref[...], preferred_element_type=jnp.float32) + bi_ref[...]
    gh = jnp.dot(h, wh_ref[...], preferred_element_type=jnp.float32) + bh_ref[...]
    r = jax.nn.sigmoid(gi[:, 0:F] + gh[:, 0:F])
    z = jax.nn.sigmoid(gi[:, F:2 * F] + gh[:, F:2 * F])
    n = jnp.tanh(gi[:, 2 * F:3 * F] + r * gh[:, 2 * F:3 * F])
    out_ref[...] = (1.0 - z) * n + z * h


def _tc_gru(parts, h, WihT, WhhT, bih2, bhh2):
    return pl.pallas_call(
        _gru_body,
        out_shape=jax.ShapeDtypeStruct((N_NODES, F), jnp.float32),
    )(parts, h, WihT, WhhT, bih2, bhh2)


def kernel(feat, edge_index, efeat, W_edge, b_edge, W_ih, W_hh, b_ih, b_hh):
    src2d = edge_index[0].reshape(NW, CPW, IR, GROUP)
    dst2d = edge_index[1].reshape(NW, CPW, IR, GROUP)
    b_edge2 = b_edge.reshape(1, FF)
    eye = jnp.eye(F, dtype=jnp.float32)
    R = jnp.repeat(eye, F, axis=1)
    T = jnp.tile(eye, (F, 1))
    WihT = W_ih.T
    WhhT = W_hh.T
    bih2 = b_ih.reshape(1, 3 * F)
    bhh2 = b_hh.reshape(1, 3 * F)
    zeros = jnp.zeros((N_PAD, F), jnp.float32)

    efeatp = efeat.reshape(N_EDGES // 8, 128)
    h = feat
    for _ in range(STEPS):
        h_src = _sc_gather(h, src2d)
        m = _tc_edge(h_src.reshape(N_EDGES // 8, 128), efeatp,
                     W_edge, b_edge2, R, T)
        parts = _sc_scatter(m.reshape(N_EDGES, F), dst2d, zeros)
        h = _tc_gru(parts, h, WihT, WhhT, bih2, bhh2)
    return h


# speedup vs baseline: 7.5534x; 1.0189x over previous
import functools

import jax
import jax.numpy as jnp
from jax import lax
from jax.experimental import pallas as pl
from jax.experimental.pallas import tpu as pltpu
from jax.experimental.pallas import tpu_sc as plsc

N_NODES = 10000
N_EDGES = 320000
F = 16
FF = F * F
STEPS = 2

GROUP = 80
NW = 32
EPW = N_EDGES // NW
CHUNK = 2000
CPW = EPW // CHUNK
IR = CHUNK // GROUP
BURSTS = IR // 5
NSUB = 16
N_PAD = 10240
NPW = N_PAD // NSUB

BE = 16000


def _sc_mesh():
    return plsc.VectorSubcoreMesh(core_axis_name="c", subcore_axis_name="s")


def _sc_gather(h, src2d):
    @functools.partial(
        pl.kernel,
        mesh=_sc_mesh(),
        out_type=jax.ShapeDtypeStruct((N_EDGES, F), jnp.float32),
        scratch_types=[
            pltpu.VMEM((IR, GROUP), jnp.int32),
            pltpu.VMEM((CHUNK, F), jnp.float32),
            pltpu.SemaphoreType.DMA,
        ],
        compiler_params=pltpu.CompilerParams(use_tc_tiling_on_sc=False),
    )
    def k(h_hbm, src_hbm, out_hbm, idx_v, rows_v, sem):
        wid = lax.axis_index("s") * 2 + lax.axis_index("c")

        def chunk_body(ci, carry):
            pltpu.sync_copy(src_hbm.at[wid, ci], idx_v)

            def burst(bi, c2):
                cps = [
                    pltpu.async_copy(
                        h_hbm.at[idx_v.at[bi * 5 + g]],
                        rows_v.at[pl.ds((bi * 5 + g) * GROUP, GROUP)],
                        sem,
                    )
                    for g in range(5)
                ]
                for cp in cps:
                    cp.wait()
                return c2

            lax.fori_loop(0, BURSTS, burst, 0)
            e0 = wid * EPW + ci * CHUNK
            pltpu.sync_copy(rows_v, out_hbm.at[pl.ds(e0, CHUNK)])
            return carry

        lax.fori_loop(0, CPW, chunk_body, 0)

    return k(h, src2d)


def _sc_scatter(m, dst2d, zeros):
    @functools.partial(
        pl.kernel,
        mesh=_sc_mesh(),
        out_type=jax.ShapeDtypeStruct((2, N_PAD, F), jnp.float32),
        scratch_types=[
            pltpu.VMEM((IR, GROUP), jnp.int32),
            pltpu.VMEM((CHUNK, F), jnp.float32),
            pltpu.VMEM_SHARED((N_PAD, F), jnp.float32),
        ],
        compiler_params=pltpu.CompilerParams(use_tc_tiling_on_sc=False),
    )
    def k(m_hbm, dst_hbm, z_hbm, out_hbm, idx_v, rows_v, acc):
        cid = lax.axis_index("c")
        sid = lax.axis_index("s")
        wid = sid * 2 + cid

        pltpu.sync_copy(z_hbm.at[pl.ds(sid * NPW, NPW)],
                        acc.at[pl.ds(sid * NPW, NPW)])
        plsc.subcore_barrier()

        def chunk_body(ci, carry):
            pltpu.sync_copy(dst_hbm.at[wid, ci], idx_v)
            e0 = wid * EPW + ci * CHUNK
            pltpu.sync_copy(m_hbm.at[pl.ds(e0, CHUNK)], rows_v)

            def burst(bi, c2):
                for g in range(5):
                    jj = bi * 5 + g
                    pltpu.sync_copy(
                        rows_v.at[pl.ds(jj * GROUP, GROUP)],
                        acc.at[idx_v.at[jj]],
                        add=True,
                    )
                return c2

            lax.fori_loop(0, BURSTS, burst, 0)
            return carry

        lax.fori_loop(0, CPW, chunk_body, 0)
        plsc.subcore_barrier()

        pltpu.sync_copy(acc.at[pl.ds(sid * NPW, NPW)],
                        out_hbm.at[cid, pl.ds(sid * NPW, NPW)])

    return k(m, dst2d, zeros)


BR = BE // 8


def _edge_body(hs_ref, ef_ref, we_ref, be_ref, r_ref, t_ref, out_ref):
    we = we_ref[...].astype(jnp.bfloat16)
    be = be_ref[...]
    r = r_ref[...].astype(jnp.bfloat16)
    t = t_ref[...].astype(jnp.bfloat16)
    for j in range(8):
        hsj = hs_ref[:, j * F:(j + 1) * F].astype(jnp.bfloat16)
        efj = ef_ref[:, j * F:(j + 1) * F].astype(jnp.bfloat16)
        w = jnp.dot(efj, we, preferred_element_type=jnp.float32) + be
        hx = jnp.dot(hsj, r, preferred_element_type=jnp.float32)
        p = (hx * w).astype(jnp.bfloat16)
        out_ref[:, j * F:(j + 1) * F] = jnp.dot(
            p, t, preferred_element_type=jnp.float32)


def _tc_edge(h_srcp, efeatp, W_edge, b_edge2, R, T):
    return pl.pallas_call(
        _edge_body,
        grid=(N_EDGES // BE,),
        in_specs=[
            pl.BlockSpec((BR, 128), lambda i: (i, 0)),
            pl.BlockSpec((BR, 128), lambda i: (i, 0)),
            pl.BlockSpec((F, FF), lambda i: (0, 0)),
            pl.BlockSpec((1, FF), lambda i: (0, 0)),
            pl.BlockSpec((F, FF), lambda i: (0, 0)),
            pl.BlockSpec((FF, F), lambda i: (0, 0)),
        ],
        out_specs=pl.BlockSpec((BR, 128), lambda i: (i, 0)),
        out_shape=jax.ShapeDtypeStruct((N_EDGES // 8, 128), jnp.float32),
    )(h_srcp, efeatp, W_edge, b_edge2, R, T)


def _gru_body(p_ref, h_ref, wi_ref, wh_ref, bi_ref, bh_ref, out_ref):
    x = p_ref[0, :N_NODES, :] + p_ref[1, :N_NODES, :]
    h = h_ref[...]
    gi = jnp.dot(x, wi_ref[...], preferred_element_type=jnp.float32) + bi_ref[...]
    gh = jnp.dot(h, wh_ref[...], preferred_element_type=jnp.float32) + bh_ref[...]
    r = jax.nn.sigmoid(gi[:, 0:F] + gh[:, 0:F])
    z = jax.nn.sigmoid(gi[:, F:2 * F] + gh[:, F:2 * F])
    n = jnp.tanh(gi[:, 2 * F:3 * F] + r * gh[:, 2 * F:3 * F])
    out_ref[...] = (1.0 - z) * n + z * h


def _tc_gru(parts, h, WihT, WhhT, bih2, bhh2):
    return pl.pallas_call(
        _gru_body,
        out_shape=jax.ShapeDtypeStruct((N_NODES, F), jnp.float32),
    )(parts, h, WihT, WhhT, bih2, bhh2)


def kernel(feat, edge_index, efeat, W_edge, b_edge, W_ih, W_hh, b_ih, b_hh):
    src2d = edge_index[0].reshape(NW, CPW, IR, GROUP)
    dst2d = edge_index[1].reshape(NW, CPW, IR, GROUP)
    b_edge2 = b_edge.reshape(1, FF)
    eye = jnp.eye(F, dtype=jnp.float32)
    R = jnp.repeat(eye, F, axis=1)
    T = jnp.tile(eye, (F, 1))
    WihT = W_ih.T
    WhhT = W_hh.T
    bih2 = b_ih.reshape(1, 3 * F)
    bhh2 = b_hh.reshape(1, 3 * F)
    zeros = jnp.zeros((N_PAD, F), jnp.float32)

    efeatp = efeat.reshape(N_EDGES // 8, 128)
    h = feat
    for _ in range(STEPS):
        h_src = _sc_gather(h, src2d)
        m = _tc_edge(h_src.reshape(N_EDGES // 8, 128), efeatp,
                     W_edge, b_edge2, R, T)
        parts = _sc_scatter(m.reshape(N_EDGES, F), dst2d, zeros)
        h = _tc_gru(parts, h, WihT, WhhT, bih2, bhh2)
    return h


# speedup vs baseline: 8.8056x; 1.1658x over previous
import functools

import jax
import jax.numpy as jnp
from jax import lax
from jax.experimental import pallas as pl
from jax.experimental.pallas import tpu as pltpu
from jax.experimental.pallas import tpu_sc as plsc

N_NODES = 10000
N_EDGES = 320000
F = 16
FF = F * F
STEPS = 2

GROUP = 80
NW = 32
EPW = N_EDGES // NW
CHUNK = 2000
CPW = EPW // CHUNK
IR = CHUNK // GROUP
BURSTS = IR // 5
NSUB = 16
N_PAD = 10240
NPW = N_PAD // NSUB

BE = 16000


def _sc_mesh():
    return plsc.VectorSubcoreMesh(core_axis_name="c", subcore_axis_name="s")


def _sc_gather(h, src2d):
    @functools.partial(
        pl.kernel,
        mesh=_sc_mesh(),
        out_type=jax.ShapeDtypeStruct((N_EDGES, F), jnp.float32),
        scratch_types=[
            pltpu.VMEM((IR, GROUP), jnp.int32),
            pltpu.VMEM((CHUNK, F), jnp.float32),
            pltpu.SemaphoreType.DMA,
        ],
        compiler_params=pltpu.CompilerParams(use_tc_tiling_on_sc=False),
    )
    def k(h_hbm, src_hbm, out_hbm, idx_v, rows_v, sem):
        wid = lax.axis_index("s") * 2 + lax.axis_index("c")

        def chunk_body(ci, carry):
            pltpu.sync_copy(src_hbm.at[wid, ci], idx_v)

            def burst(bi, c2):
                cps = [
                    pltpu.async_copy(
                        h_hbm.at[idx_v.at[bi * 5 + g]],
                        rows_v.at[pl.ds((bi * 5 + g) * GROUP, GROUP)],
                        sem,
                    )
                    for g in range(5)
                ]
                for cp in cps:
                    cp.wait()
                return c2

            lax.fori_loop(0, BURSTS, burst, 0)
            e0 = wid * EPW + ci * CHUNK
            pltpu.sync_copy(rows_v, out_hbm.at[pl.ds(e0, CHUNK)])
            return carry

        lax.fori_loop(0, CPW, chunk_body, 0)

    return k(h, src2d)


def _sc_scatter(m, dst2d, zeros):
    @functools.partial(
        pl.kernel,
        mesh=_sc_mesh(),
        out_type=jax.ShapeDtypeStruct((2, N_PAD, F), jnp.float32),
        scratch_types=[
            pltpu.VMEM((IR, GROUP), jnp.int32),
            pltpu.VMEM((CHUNK, F), jnp.float32),
            pltpu.VMEM_SHARED((N_PAD, F), jnp.float32),
        ],
        compiler_params=pltpu.CompilerParams(use_tc_tiling_on_sc=False),
    )
    def k(m_hbm, dst_hbm, z_hbm, out_hbm, idx_v, rows_v, acc):
        cid = lax.axis_index("c")
        sid = lax.axis_index("s")
        wid = sid * 2 + cid

        pltpu.sync_copy(z_hbm.at[pl.ds(sid * NPW, NPW)],
                        acc.at[pl.ds(sid * NPW, NPW)])
        plsc.subcore_barrier()

        def chunk_body(ci, carry):
            pltpu.sync_copy(dst_hbm.at[wid, ci], idx_v)
            e0 = wid * EPW + ci * CHUNK
            pltpu.sync_copy(m_hbm.at[pl.ds(e0, CHUNK)], rows_v)

            def burst(bi, c2):
                for g in range(5):
                    jj = bi * 5 + g
                    pltpu.sync_copy(
                        rows_v.at[pl.ds(jj * GROUP, GROUP)],
                        acc.at[idx_v.at[jj]],
                        add=True,
                    )
                return c2

            lax.fori_loop(0, BURSTS, burst, 0)
            return carry

        lax.fori_loop(0, CPW, chunk_body, 0)
        plsc.subcore_barrier()

        pltpu.sync_copy(acc.at[pl.ds(sid * NPW, NPW)],
                        out_hbm.at[cid, pl.ds(sid * NPW, NPW)])

    return k(m, dst2d, zeros)


BR = BE // 8


def _edge_body(hs_ref, ef_ref, ww_ref, wh_ref, be_ref, out_ref):
    hs = hs_ref[...].astype(jnp.bfloat16)
    ef = ef_ref[...].astype(jnp.bfloat16)
    acc = jnp.zeros((BR, 128), jnp.float32)
    for p in range(8):
        w = jnp.dot(ef, ww_ref[128 * p:128 * (p + 1), :],
                    preferred_element_type=jnp.float32) + be_ref[p, :]
        hx = jnp.dot(hs, wh_ref[128 * p:128 * (p + 1), :],
                     preferred_element_type=jnp.float32)
        pr = hx * w
        acc = acc + pr[:, :128] + pr[:, 128:]
    out_ref[...] = acc


def _tc_edge(h_srcp, efeatp, WW, WH, BE2):
    return pl.pallas_call(
        _edge_body,
        grid=(N_EDGES // BE,),
        in_specs=[
            pl.BlockSpec((BR, 128), lambda i: (i, 0)),
            pl.BlockSpec((BR, 128), lambda i: (i, 0)),
            pl.BlockSpec((1024, 256), lambda i: (0, 0)),
            pl.BlockSpec((1024, 256), lambda i: (0, 0)),
            pl.BlockSpec((8, 256), lambda i: (0, 0)),
        ],
        out_specs=pl.BlockSpec((BR, 128), lambda i: (i, 0)),
        out_shape=jax.ShapeDtypeStruct((N_EDGES // 8, 128), jnp.float32),
    )(h_srcp, efeatp, WW, WH, BE2)


def _gru_body(p_ref, h_ref, wi_ref, wh_ref, bi_ref, bh_ref, out_ref):
    x = p_ref[0, :N_NODES, :] + p_ref[1, :N_NODES, :]
    h = h_ref[...]
    gi = jnp.dot(x, wi_ref[...], preferred_element_type=jnp.float32) + bi_ref[...]
    gh = jnp.dot(h, wh_ref[...], preferred_element_type=jnp.float32) + bh_ref[...]
    r = jax.nn.sigmoid(gi[:, 0:F] + gh[:, 0:F])
    z = jax.nn.sigmoid(gi[:, F:2 * F] + gh[:, F:2 * F])
    n = jnp.tanh(gi[:, 2 * F:3 * F] + r * gh[:, 2 * F:3 * F])
    out_ref[...] = (1.0 - z) * n + z * h


def _tc_gru(parts, h, WihT, WhhT, bih2, bhh2):
    return pl.pallas_call(
        _gru_body,
        out_shape=jax.ShapeDtypeStruct((N_NODES, F), jnp.float32),
    )(parts, h, WihT, WhhT, bih2, bhh2)


def kernel(feat, edge_index, efeat, W_edge, b_edge, W_ih, W_hh, b_ih, b_hh):
    src2d = edge_index[0].reshape(NW, CPW, IR, GROUP)
    dst2d = edge_index[1].reshape(NW, CPW, IR, GROUP)
    eye8 = jnp.eye(8, dtype=jnp.float32)
    wk = W_edge.reshape(F, F, F)
    bd_w = jnp.einsum('ab,fko->kafbo', eye8, wk).reshape(F, 128, 128)
    WW = jnp.concatenate([bd_w[0::2], bd_w[1::2]], axis=2).reshape(1024, 256)
    sel = jnp.eye(F, dtype=jnp.float32)[:, :, None] * jnp.ones(
        (1, 1, F), jnp.float32)
    bd_h = jnp.einsum('ab,kio->kaibo', eye8, sel).reshape(F, 128, 128)
    WH = jnp.concatenate([bd_h[0::2], bd_h[1::2]], axis=2).reshape(1024, 256)
    bk = jnp.tile(b_edge.reshape(F, 1, F), (1, 8, 1)).reshape(F, 128)
    BE2 = jnp.concatenate([bk[0::2], bk[1::2]], axis=1)
    WW = WW.astype(jnp.bfloat16)
    WH = WH.astype(jnp.bfloat16)
    WihT = W_ih.T
    WhhT = W_hh.T
    bih2 = b_ih.reshape(1, 3 * F)
    bhh2 = b_hh.reshape(1, 3 * F)
    zeros = jnp.zeros((N_PAD, F), jnp.float32)

    efeatp = efeat.reshape(N_EDGES // 8, 128)
    h = feat
    for _ in range(STEPS):
        h_src = _sc_gather(h, src2d)
        m = _tc_edge(h_src.reshape(N_EDGES // 8, 128), efeatp, WW, WH, BE2)
        parts = _sc_scatter(m.reshape(N_EDGES, F), dst2d, zeros)
        h = _tc_gru(parts, h, WihT, WhhT, bih2, bhh2)
    return h


# speedup vs baseline: 9.3375x; 1.0604x over previous
import functools

import jax
import jax.numpy as jnp
from jax import lax
from jax.experimental import pallas as pl
from jax.experimental.pallas import tpu as pltpu
from jax.experimental.pallas import tpu_sc as plsc

N_NODES = 10000
N_EDGES = 320000
F = 16
FF = F * F
STEPS = 2

EH = N_EDGES // 2
NW = 32
NSUB = 16
N_PAD = 10240
NPW = N_PAD // NSUB

EPW = EH // NW
CHUNK = 1000
CPW = EPW // CHUNK
GROUP = 40
IR = CHUNK // GROUP
BURSTS = IR // 5

BE = 16000
BR = BE // 8
NPR = N_NODES // 8
NPADR = N_PAD // 8


def _sc_mesh():
    return plsc.VectorSubcoreMesh(core_axis_name="c", subcore_axis_name="s")


def _sc_gather(h, src4d):
    @functools.partial(
        pl.kernel,
        mesh=_sc_mesh(),
        out_type=jax.ShapeDtypeStruct((EH, F), jnp.float32),
        scratch_types=[
            pltpu.VMEM((IR, GROUP), jnp.int32),
            pltpu.VMEM((CHUNK, F), jnp.float32),
            pltpu.SemaphoreType.DMA,
        ],
        compiler_params=pltpu.CompilerParams(use_tc_tiling_on_sc=False),
    )
    def k(h_hbm, src_hbm, out_hbm, idx_v, rows_v, sem):
        wid = lax.axis_index("s") * 2 + lax.axis_index("c")

        def chunk_body(ci, carry):
            pltpu.sync_copy(src_hbm.at[wid, ci], idx_v)

            def burst(bi, c2):
                cps = [
                    pltpu.async_copy(
                        h_hbm.at[idx_v.at[bi * 5 + g]],
                        rows_v.at[pl.ds((bi * 5 + g) * GROUP, GROUP)],
                        sem,
                    )
                    for g in range(5)
                ]
                for cp in cps:
                    cp.wait()
                return c2

            lax.fori_loop(0, BURSTS, burst, 0)
            e0 = wid * EPW + ci * CHUNK
            pltpu.sync_copy(rows_v, out_hbm.at[pl.ds(e0, CHUNK)])
            return carry

        lax.fori_loop(0, CPW, chunk_body, 0)

    return k(h, src4d)


def _sc_scatter(m, dst4d, zeros):
    @functools.partial(
        pl.kernel,
        mesh=_sc_mesh(),
        out_type=jax.ShapeDtypeStruct((2, N_PAD, F), jnp.float32),
        scratch_types=[
            pltpu.VMEM((IR, GROUP), jnp.int32),
            pltpu.VMEM((CHUNK, F), jnp.float32),
            pltpu.VMEM_SHARED((N_PAD, F), jnp.float32),
        ],
        compiler_params=pltpu.CompilerParams(use_tc_tiling_on_sc=False),
    )
    def k(m_hbm, dst_hbm, z_hbm, out_hbm, idx_v, rows_v, acc):
        cid = lax.axis_index("c")
        sid = lax.axis_index("s")
        wid = sid * 2 + cid

        pltpu.sync_copy(z_hbm.at[pl.ds(sid * NPW, NPW)],
                        acc.at[pl.ds(sid * NPW, NPW)])
        plsc.subcore_barrier()

        def chunk_body(ci, carry):
            pltpu.sync_copy(dst_hbm.at[wid, ci], idx_v)
            e0 = wid * EPW + ci * CHUNK
            pltpu.sync_copy(m_hbm.at[pl.ds(e0, CHUNK)], rows_v)

            def burst(bi, c2):
                for g in range(5):
                    jj = bi * 5 + g
                    pltpu.sync_copy(
                        rows_v.at[pl.ds(jj * GROUP, GROUP)],
                        acc.at[idx_v.at[jj]],
                        add=True,
                    )
                return c2

            lax.fori_loop(0, BURSTS, burst, 0)
            return carry

        lax.fori_loop(0, CPW, chunk_body, 0)
        plsc.subcore_barrier()

        pltpu.sync_copy(acc.at[pl.ds(sid * NPW, NPW)],
                        out_hbm.at[cid, pl.ds(sid * NPW, NPW)])

    return k(m, dst4d, zeros)


def _edge_body(hs_ref, ef_ref, ww_ref, wh_ref, be_ref, out_ref):
    hs = hs_ref[...].astype(jnp.bfloat16)
    ef = ef_ref[...].astype(jnp.bfloat16)
    acc = jnp.zeros((BR, 128), jnp.float32)
    for p in range(8):
        w = jnp.dot(ef, ww_ref[128 * p:128 * (p + 1), :],
                    preferred_element_type=jnp.float32) + be_ref[p, :]
        hx = jnp.dot(hs, wh_ref[128 * p:128 * (p + 1), :],
                     preferred_element_type=jnp.float32)
        pr = hx * w
        acc = acc + pr[:, :128] + pr[:, 128:]
    out_ref[...] = acc


def _tc_edge(h_srcp, efp, WW, WH, BE2, ef_base):
    return pl.pallas_call(
        _edge_body,
        grid=(EH // BE,),
        in_specs=[
            pl.BlockSpec((BR, 128), lambda i: (i, 0)),
            pl.BlockSpec((BR, 128), lambda i, b=ef_base: (i + b, 0)),
            pl.BlockSpec((1024, 256), lambda i: (0, 0)),
            pl.BlockSpec((1024, 256), lambda i: (0, 0)),
            pl.BlockSpec((8, 256), lambda i: (0, 0)),
        ],
        out_specs=pl.BlockSpec((BR, 128), lambda i: (i, 0)),
        out_shape=jax.ShapeDtypeStruct((EH // 8, 128), jnp.float32),
    )(h_srcp, efp, WW, WH, BE2)


def _gru_body(pa_ref, pb_ref, h_ref, wi_ref, wh_ref, bi_ref, bh_ref, out_ref):
    wi = wi_ref[...]
    wh = wh_ref[...]
    bi = bi_ref[...]
    bh = bh_ref[...]
    for j in range(8):
        sl = slice(j * F, (j + 1) * F)
        x = (pa_ref[0, :NPR, sl] + pa_ref[1, :NPR, sl]
             + pb_ref[0, :NPR, sl] + pb_ref[1, :NPR, sl])
        h = h_ref[:, sl]
        gi = jnp.dot(x, wi, preferred_element_type=jnp.float32) + bi
        gh = jnp.dot(h, wh, preferred_element_type=jnp.float32) + bh
        r = jax.nn.sigmoid(gi[:, 0:F] + gh[:, 0:F])
        z = jax.nn.sigmoid(gi[:, F:2 * F] + gh[:, F:2 * F])
        n = jnp.tanh(gi[:, 2 * F:3 * F] + r * gh[:, 2 * F:3 * F])
        out_ref[:, sl] = (1.0 - z) * n + z * h


def _tc_gru(partsA, partsB, hp, WihT, WhhT, bih2, bhh2):
    return pl.pallas_call(
        _gru_body,
        out_shape=jax.ShapeDtypeStruct((NPR, 128), jnp.float32),
    )(partsA, partsB, hp, WihT, WhhT, bih2, bhh2)


def kernel(feat, edge_index, efeat, W_edge, b_edge, W_ih, W_hh, b_ih, b_hh):
    srcA = edge_index[0, :EH].reshape(NW, CPW, IR, GROUP)
    srcB = edge_index[0, EH:].reshape(NW, CPW, IR, GROUP)
    dstA = edge_index[1, :EH].reshape(NW, CPW, IR, GROUP)
    dstB = edge_index[1, EH:].reshape(NW, CPW, IR, GROUP)
    eye8 = jnp.eye(8, dtype=jnp.float32)
    wk = W_edge.reshape(F, F, F)
    bd_w = jnp.einsum('ab,fko->kafbo', eye8, wk).reshape(F, 128, 128)
    WW = jnp.concatenate([bd_w[0::2], bd_w[1::2]], axis=2).reshape(1024, 256)
    sel = jnp.eye(F, dtype=jnp.float32)[:, :, None] * jnp.ones(
        (1, 1, F), jnp.float32)
    bd_h = jnp.einsum('ab,kio->kaibo', eye8, sel).reshape(F, 128, 128)
    WH = jnp.concatenate([bd_h[0::2], bd_h[1::2]], axis=2).reshape(1024, 256)
    bk = jnp.tile(b_edge.reshape(F, 1, F), (1, 8, 1)).reshape(F, 128)
    BE2 = jnp.concatenate([bk[0::2], bk[1::2]], axis=1)
    WW = WW.astype(jnp.bfloat16)
    WH = WH.astype(jnp.bfloat16)
    WihT = W_ih.T
    WhhT = W_hh.T
    bih2 = b_ih.reshape(1, 3 * F)
    bhh2 = b_hh.reshape(1, 3 * F)
    zeros = jnp.zeros((N_PAD, F), jnp.float32)

    efp = efeat.reshape(N_EDGES // 8, 128)
    hp = feat.reshape(NPR, 128)
    for _ in range(STEPS):
        h = hp.reshape(N_NODES, F)
        gA = _sc_gather(h, srcA)
        gB = _sc_gather(h, srcB)
        mA = _tc_edge(gA.reshape(EH // 8, 128), efp, WW, WH, BE2, 0)
        mB = _tc_edge(gB.reshape(EH // 8, 128), efp, WW, WH, BE2, EH // BE)
        pA = _sc_scatter(mA.reshape(EH, F), dstA, zeros)
        pB = _sc_scatter(mB.reshape(EH, F), dstB, zeros)
        hp = _tc_gru(pA.reshape(2, NPADR, 128), pB.reshape(2, NPADR, 128),
                     hp, WihT, WhhT, bih2, bhh2)
    return hp.reshape(N_NODES, F)
